# Initial kernel scaffold; baseline (speedup 1.0000x reference)
import jax
import jax.numpy as jnp
from jax.experimental import pallas as pl


def kernel(token_ids, embedding):
    raise NotImplementedError("write your pallas kernel here")


# speedup vs baseline: 1.8442x; 1.8442x over previous
import functools

import jax
import jax.numpy as jnp
from jax import lax
from jax.experimental import pallas as pl
from jax.experimental.pallas import tpu as pltpu
from jax.experimental.pallas import tpu_sc as plsc

BATCH = 16384
HIST = 50
EMBEDDING_DIM = 64

_INFO = plsc.get_sparse_core_info()
_NC = _INFO.num_cores
_NS = _INFO.num_subcores
_NW = _NC * _NS

_B = BATCH * HIST
_IDXW = 128
_ROWS_PER_CHUNK = 1024
_IDX_ROWS_PER_CHUNK = _ROWS_PER_CHUNK // _IDXW
_ROWS_PER_WORKER = _B // _NW
_CHUNKS = _ROWS_PER_WORKER // _ROWS_PER_CHUNK


def _gather_body(idx_hbm, table_hbm, out_hbm, idx_v, rows_v, idx_sem, gat_sem):
    wid = lax.axis_index("s") * _NC + lax.axis_index("c")
    base_row = wid * _ROWS_PER_WORKER
    base_irow = base_row // _IDXW

    @pl.loop(0, _CHUNKS)
    def _chunk(g):
        irow = pl.multiple_of(base_irow + g * _IDX_ROWS_PER_CHUNK, 8)
        pltpu.async_copy(
            idx_hbm.at[pl.ds(irow, _IDX_ROWS_PER_CHUNK)], idx_v, idx_sem
        ).wait()
        for j in range(_IDX_ROWS_PER_CHUNK):
            pltpu.async_copy(
                table_hbm.at[idx_v.at[j]],
                rows_v.at[pl.ds(j * _IDXW, _IDXW)],
                gat_sem,
            )
        for j in range(_IDX_ROWS_PER_CHUNK):
            pltpu.make_async_copy(
                table_hbm.at[idx_v.at[j]],
                rows_v.at[pl.ds(j * _IDXW, _IDXW)],
                gat_sem,
            ).wait()
        row = base_row + g * _ROWS_PER_CHUNK
        pltpu.sync_copy(rows_v, out_hbm.at[pl.ds(row, _ROWS_PER_CHUNK)])


@jax.jit
def _lookup(idx2d, table):
    mesh = plsc.VectorSubcoreMesh(core_axis_name="c", subcore_axis_name="s")
    fn = pl.kernel(
        _gather_body,
        out_type=jax.ShapeDtypeStruct((_B, EMBEDDING_DIM), jnp.float32),
        mesh=mesh,
        scratch_types=[
            pltpu.VMEM((_IDX_ROWS_PER_CHUNK, _IDXW), jnp.int32),
            pltpu.VMEM((_ROWS_PER_CHUNK, EMBEDDING_DIM), jnp.float32),
            pltpu.SemaphoreType.DMA,
            pltpu.SemaphoreType.DMA,
        ],
        compiler_params=pltpu.CompilerParams(use_tc_tiling_on_sc=False),
    )
    return fn(idx2d, table)


def kernel(token_ids, embedding):
    idx2d = token_ids.astype(jnp.int32).reshape(_B // _IDXW, _IDXW)
    out = _lookup(idx2d, embedding)
    return out.reshape(BATCH, HIST, EMBEDDING_DIM)


# speedup vs baseline: 1.8636x; 1.0105x over previous
import jax
import jax.numpy as jnp
from jax import lax
from jax.experimental import pallas as pl
from jax.experimental.pallas import tpu as pltpu
from jax.experimental.pallas import tpu_sc as plsc

BATCH = 16384
HIST = 50
EMBEDDING_DIM = 64

_INFO = plsc.get_sparse_core_info()
_NC = _INFO.num_cores
_NS = _INFO.num_subcores
_NW = _NC * _NS

_B = BATCH * HIST
_IDXW = 128
_ROWS_PER_WORKER = _B // _NW
_IDX_ROWS = _ROWS_PER_WORKER // _IDXW
_ROWS_PER_CHUNK = 640
_GATHERS_PER_CHUNK = _ROWS_PER_CHUNK // _IDXW
_CHUNKS = _ROWS_PER_WORKER // _ROWS_PER_CHUNK
_NBUF = 2


def _gather_body(idx_hbm, table_hbm, out_hbm, idx_v, rows_v, gat_sems, out_sems):
    wid = lax.axis_index("s") * _NC + lax.axis_index("c")
    base_row = wid * _ROWS_PER_WORKER
    base_irow = pl.multiple_of(wid * _IDX_ROWS, 8)

    pltpu.sync_copy(idx_hbm.at[pl.ds(base_irow, _IDX_ROWS)], idx_v)

    def fire_gathers(g, b):
        for j in range(_GATHERS_PER_CHUNK):
            pltpu.async_copy(
                table_hbm.at[idx_v.at[g * _GATHERS_PER_CHUNK + j]],
                rows_v.at[b].at[pl.ds(j * _IDXW, _IDXW)],
                gat_sems[b],
            )

    def drain_gathers(g, b):
        for j in range(_GATHERS_PER_CHUNK):
            pltpu.make_async_copy(
                table_hbm.at[idx_v.at[g * _GATHERS_PER_CHUNK + j]],
                rows_v.at[b].at[pl.ds(j * _IDXW, _IDXW)],
                gat_sems[b],
            ).wait()

    def out_copy(g, b):
        return pltpu.async_copy(
            rows_v.at[b],
            out_hbm.at[pl.ds(base_row + g * _ROWS_PER_CHUNK, _ROWS_PER_CHUNK)],
            out_sems[b],
        )

    def wait_out(g, b):
        pltpu.make_async_copy(
            rows_v.at[b],
            out_hbm.at[pl.ds(base_row + g * _ROWS_PER_CHUNK, _ROWS_PER_CHUNK)],
            out_sems[b],
        ).wait()

    for b in range(_NBUF):
        fire_gathers(b, b)

    @pl.loop(0, _CHUNKS, step=_NBUF)
    def _chunk(g0):
        for b in range(_NBUF):
            drain_gathers(g0 + b, b)
            out_copy(g0 + b, b)
        for b in range(_NBUF):
            nxt = g0 + b + _NBUF

            @pl.when(nxt < _CHUNKS)
            def _():
                wait_out(g0 + b, b)
                fire_gathers(nxt, b)

    for b in range(_NBUF):
        wait_out(_CHUNKS - _NBUF + b, b)


@jax.jit
def _lookup(idx2d, table):
    mesh = plsc.VectorSubcoreMesh(core_axis_name="c", subcore_axis_name="s")
    fn = pl.kernel(
        _gather_body,
        out_type=jax.ShapeDtypeStruct((_B, EMBEDDING_DIM), jnp.float32),
        mesh=mesh,
        scratch_types=[
            pltpu.VMEM((_IDX_ROWS, _IDXW), jnp.int32),
            pltpu.VMEM((_NBUF, _ROWS_PER_CHUNK, EMBEDDING_DIM), jnp.float32),
            [pltpu.SemaphoreType.DMA] * _NBUF,
            [pltpu.SemaphoreType.DMA] * _NBUF,
        ],
        compiler_params=pltpu.CompilerParams(use_tc_tiling_on_sc=False),
    )
    return fn(idx2d, table)


def kernel(token_ids, embedding):
    idx2d = token_ids.astype(jnp.int32).reshape(_B // _IDXW, _IDXW)
    out = _lookup(idx2d, embedding)
    return out.reshape(BATCH, HIST, EMBEDDING_DIM)
